# baseline (speedup 1.0000x reference)
import jax
import jax.numpy as jnp
from jax.experimental import pallas as pl
from jax.experimental.pallas import tpu as pltpu

B = 16
HW = 1024
K = 1024
D = 256


def _vq_kernel(z_ref, w_ref, out_ref, wp_ref, wsq_ref):
    @pl.when(pl.program_id(0) == 0)
    def _prep():
        w2 = w_ref[...] + w_ref[...]
        p_row = jax.lax.broadcasted_iota(jnp.int32, (K, K), 0)
        src = jax.lax.broadcasted_iota(jnp.int32, (K, K), 1)
        jperm = (((p_row & 127) ^ 127) << 3) | (p_row >> 7)
        P = (src == jperm).astype(jnp.float32)
        wp = jax.lax.dot_general(
            P, w2, (((1,), (0,)), ((), ())),
            preferred_element_type=jnp.float32)
        wp_ref[...] = wp
        wsq_ref[0, :] = jnp.sum(wp * wp, axis=1) * 0.25

    z = z_ref[0]
    w2 = wp_ref[...]
    wsq = wsq_ref[0, :]
    zsq = jnp.sum(z * z, axis=1)
    mm2 = jax.lax.dot_general(
        z, w2, (((1,), (1,)), ((), ())),
        preferred_element_type=jnp.float32)
    d = (zsq[:, None] + wsq[None, :]) - mm2
    idx = jnp.argmin(d, axis=1).astype(jnp.int32)
    out_ref[0, :, 0] = (((idx & 127) ^ 127) << 3) | (idx >> 7)


def kernel(z_e, W):
    TB = 512
    G = (B * HW) // TB
    zg = z_e.reshape(G, TB, D)
    out = pl.pallas_call(
        _vq_kernel,
        grid=(G,),
        in_specs=[
            pl.BlockSpec((1, TB, D), lambda b: (b, 0, 0)),
            pl.BlockSpec((K, D), lambda b: (0, 0)),
        ],
        out_specs=pl.BlockSpec((1, TB, 1), lambda b: (b, 0, 0)),
        out_shape=jax.ShapeDtypeStruct((G, TB, 1), jnp.int32),
        scratch_shapes=[
            pltpu.VMEM((K, D), jnp.float32),
            pltpu.VMEM((1, K), jnp.float32),
        ],
        compiler_params=pltpu.CompilerParams(
            dimension_semantics=("arbitrary",)),
    )(zg, W)
    return out.reshape(B, HW)


# speedup vs baseline: 1.3988x; 1.3988x over previous
import jax
import jax.numpy as jnp
from jax.experimental import pallas as pl
from jax.experimental.pallas import tpu as pltpu

B = 16
HW = 1024
K = 1024
D = 256


def _vq_kernel(z_ref, w_ref, out_ref, wp_ref, wsq_ref):
    @pl.when(pl.program_id(0) == 0)
    def _prep():
        w2 = w_ref[...] + w_ref[...]
        p_row = jax.lax.broadcasted_iota(jnp.int32, (K, K), 0)
        src = jax.lax.broadcasted_iota(jnp.int32, (K, K), 1)
        jperm = (((p_row & 127) ^ 127) << 3) | (p_row >> 7)
        P = (src == jperm).astype(jnp.float32)
        wp = jax.lax.dot_general(
            P, w2, (((1,), (0,)), ((), ())),
            preferred_element_type=jnp.float32)
        wp_ref[...] = wp
        wsq_ref[0, :] = jnp.sum(wp * wp, axis=1) * 0.25

    z = z_ref[0]
    w2 = wp_ref[...]
    wsq = wsq_ref[0, :]
    zsq = jnp.sum(z * z, axis=1)
    mm2 = jax.lax.dot_general(
        z, w2, (((1,), (1,)), ((), ())),
        preferred_element_type=jnp.float32)
    d = (zsq[:, None] + wsq[None, :]) - mm2
    idx = jnp.argmin(d, axis=1).astype(jnp.int32)
    out_ref[0, :, 0] = (((idx & 127) ^ 127) << 3) | (idx >> 7)


def kernel(z_e, W):
    TB = 2048
    G = (B * HW) // TB
    zg = z_e.reshape(G, TB, D)
    out = pl.pallas_call(
        _vq_kernel,
        grid=(G,),
        in_specs=[
            pl.BlockSpec((1, TB, D), lambda b: (b, 0, 0)),
            pl.BlockSpec((K, D), lambda b: (0, 0)),
        ],
        out_specs=pl.BlockSpec((1, TB, 1), lambda b: (b, 0, 0)),
        out_shape=jax.ShapeDtypeStruct((G, TB, 1), jnp.int32),
        scratch_shapes=[
            pltpu.VMEM((K, D), jnp.float32),
            pltpu.VMEM((1, K), jnp.float32),
        ],
        compiler_params=pltpu.CompilerParams(
            dimension_semantics=("arbitrary",)),
    )(zg, W)
    return out.reshape(B, HW)
